# Initial kernel scaffold; baseline (speedup 1.0000x reference)
import jax
import jax.numpy as jnp
from jax.experimental import pallas as pl


def kernel(x, edge_index, g, A_k, D, Kindices, de, M, I, W1, b1, Wmlp, bmlp, W2, b2):
    raise NotImplementedError("write your pallas kernel here")


# speedup vs baseline: 39.1850x; 39.1850x over previous
import functools

import jax
import jax.numpy as jnp
from jax import lax
from jax.experimental import pallas as pl
from jax.experimental.pallas import tpu as pltpu
from jax.experimental.pallas import tpu_sc as plsc

N = 10000
E = 320000
IN_DIM = 128
HID = 32
NC_OUT = 16

NCORES = 2
NSUB = 16
NW = NCORES * NSUB
EPT = E // NW
BATCH = 125
NB = EPT // BATCH
NPAD = 10240
NODES_PER_SUB = NPAD // NSUB

_MESH = plsc.VectorSubcoreMesh(
    core_axis_name="c", subcore_axis_name="s",
    num_cores=NCORES, num_subcores=NSUB)
_SC_PARAMS = pltpu.CompilerParams(use_tc_tiling_on_sc=False)


@functools.partial(
    pl.kernel,
    out_type=jax.ShapeDtypeStruct((NCORES * NPAD, 8), jnp.float32),
    mesh=_MESH,
    compiler_params=_SC_PARAMS,
    scratch_types=[
        pltpu.VMEM((NB, BATCH), jnp.int32),
        pltpu.VMEM((BATCH, 8), jnp.float32),
        pltpu.VMEM_SHARED((NPAD, 8), jnp.float32),
    ],
)
def _deg_kernel(dst_hbm, ones_hbm, zeros_hbm, out_hbm, didx, ones_v, acc):
    c = lax.axis_index("c")
    s = lax.axis_index("s")
    w = c * NSUB + s
    pltpu.sync_copy(dst_hbm.at[pl.ds(w * NB, NB)], didx)
    pltpu.sync_copy(ones_hbm, ones_v)
    base = s * NODES_PER_SUB
    pltpu.sync_copy(zeros_hbm.at[pl.ds(base, NODES_PER_SUB)],
                    acc.at[pl.ds(base, NODES_PER_SUB)])
    plsc.subcore_barrier()

    def body(i, carry):
        pltpu.sync_copy(ones_v, acc.at[didx.at[i]], add=True)
        return carry

    lax.fori_loop(0, NB, body, 0)
    plsc.subcore_barrier()
    pltpu.sync_copy(acc.at[pl.ds(base, NODES_PER_SUB)],
                    out_hbm.at[pl.ds(c * NPAD + base, NODES_PER_SUB)])


def _make_conv_scatter(feat):
    @functools.partial(
        pl.kernel,
        out_type=jax.ShapeDtypeStruct((NCORES * NPAD, feat), jnp.float32),
        mesh=_MESH,
        compiler_params=_SC_PARAMS,
        scratch_types=[
            pltpu.VMEM((NB, BATCH), jnp.int32),
            pltpu.VMEM((NB, BATCH), jnp.int32),
            pltpu.VMEM((2, BATCH, feat), jnp.float32),
            pltpu.VMEM_SHARED((NPAD, feat), jnp.float32),
            pltpu.SemaphoreType.DMA,
            pltpu.SemaphoreType.DMA,
        ],
    )
    def conv_scatter(hs_hbm, src_hbm, dst_hbm, zeros_hbm, out_hbm,
                     sidx, didx, rows, acc, sem0, sem1):
        c = lax.axis_index("c")
        s = lax.axis_index("s")
        w = c * NSUB + s
        pltpu.sync_copy(src_hbm.at[pl.ds(w * NB, NB)], sidx)
        pltpu.sync_copy(dst_hbm.at[pl.ds(w * NB, NB)], didx)
        base = s * NODES_PER_SUB
        pltpu.sync_copy(zeros_hbm.at[pl.ds(base, NODES_PER_SUB)],
                        acc.at[pl.ds(base, NODES_PER_SUB)])
        plsc.subcore_barrier()

        pltpu.async_copy(hs_hbm.at[sidx.at[0]], rows.at[0], sem0)

        def body(i, carry):
            b0 = 2 * i
            pltpu.async_copy(hs_hbm.at[sidx.at[b0 + 1]], rows.at[1], sem1)
            pltpu.make_async_copy(hs_hbm.at[sidx.at[0]], rows.at[0],
                                  sem0).wait()
            pltpu.sync_copy(rows.at[0], acc.at[didx.at[b0]], add=True)

            @pl.when(b0 + 2 < NB)
            def _():
                pltpu.async_copy(hs_hbm.at[sidx.at[b0 + 2]], rows.at[0],
                                 sem0)

            pltpu.make_async_copy(hs_hbm.at[sidx.at[0]], rows.at[1],
                                  sem1).wait()
            pltpu.sync_copy(rows.at[1], acc.at[didx.at[b0 + 1]], add=True)
            return carry

        lax.fori_loop(0, NB // 2, body, 0)

        plsc.subcore_barrier()
        pltpu.sync_copy(acc.at[pl.ds(base, NODES_PER_SUB)],
                        out_hbm.at[pl.ds(c * NPAD + base, NODES_PER_SUB)])

    return conv_scatter


_conv32 = _make_conv_scatter(HID)
_conv16 = _make_conv_scatter(NC_OUT)


def _mm_body(x_ref, w_ref, o_ref):
    o_ref[...] = jnp.dot(x_ref[...], w_ref[...],
                         preferred_element_type=jnp.float32)


def _scale_body(degp_ref, h_ref, hs_ref, dinv_ref):
    deg8 = degp_ref[0] + degp_ref[1] + 1.0
    dinv8 = lax.rsqrt(deg8)
    dinv_ref[...] = dinv8
    hs_ref[...] = h_ref[...] * dinv8[:, :1]


def _combine_body(p_ref, h_ref, hmlp_ref, dinv_ref, w2_ref, b1_ref, bmlp_ref,
                  g2_ref, g2s_ref):
    d1 = dinv_ref[:, :1]
    out1 = d1 * (p_ref[0] + p_ref[1]) + d1 * d1 * h_ref[...] + b1_ref[...]
    x2 = jnp.maximum(out1, 0.0) + hmlp_ref[...] + bmlp_ref[...]
    g2 = jnp.dot(x2, w2_ref[...], preferred_element_type=jnp.float32)
    g2_ref[...] = g2
    g2s_ref[...] = g2 * d1


def _final_body(q_ref, g2_ref, dinv_ref, b2_ref, o_ref):
    d1 = dinv_ref[:, :1]
    out2 = d1 * (q_ref[0] + q_ref[1]) + d1 * d1 * g2_ref[...] + b2_ref[...]
    m = jnp.max(out2, axis=1, keepdims=True)
    e = jnp.exp(out2 - m)
    o_ref[...] = e / jnp.sum(e, axis=1, keepdims=True)


_mm = pl.pallas_call(
    _mm_body, out_shape=jax.ShapeDtypeStruct((N, 2 * HID), jnp.float32))

_scale = pl.pallas_call(
    _scale_body,
    out_shape=[jax.ShapeDtypeStruct((N, HID), jnp.float32),
               jax.ShapeDtypeStruct((N, 8), jnp.float32)])

_combine = pl.pallas_call(
    _combine_body,
    out_shape=[jax.ShapeDtypeStruct((N, NC_OUT), jnp.float32),
               jax.ShapeDtypeStruct((N, NC_OUT), jnp.float32)])

_final = pl.pallas_call(
    _final_body, out_shape=jax.ShapeDtypeStruct((N, NC_OUT), jnp.float32))


def kernel(x, edge_index, g, A_k, D, Kindices, de, M, I,
           W1, b1, Wmlp, bmlp, W2, b2):
    f32 = jnp.float32
    src = edge_index[0].reshape(NW * NB, BATCH)
    dst = edge_index[1].reshape(NW * NB, BATCH)
    ones8 = jnp.ones((BATCH, 8), f32)
    zeros8 = jnp.zeros((NPAD, 8), f32)
    zeros32 = jnp.zeros((NPAD, HID), f32)
    zeros16 = jnp.zeros((NPAD, NC_OUT), f32)

    degp = _deg_kernel(dst, ones8, zeros8).reshape(NCORES, NPAD, 8)[:, :N]
    hh = _mm(x, jnp.concatenate([W1, Wmlp], axis=1))
    h = hh[:, :HID]
    hmlp = hh[:, HID:]
    hs, dinv8 = _scale(degp, h)
    p = _conv32(hs, src, dst, zeros32).reshape(NCORES, NPAD, HID)[:, :N]
    g2, g2s = _combine(p, h, hmlp, dinv8, W2,
                       b1.reshape(1, HID), bmlp.reshape(1, HID))
    q = _conv16(g2s, src, dst, zeros16).reshape(NCORES, NPAD, NC_OUT)[:, :N]
    return _final(q, g2, dinv8, b2.reshape(1, NC_OUT))


# speedup vs baseline: 51.4735x; 1.3136x over previous
import functools

import jax
import jax.numpy as jnp
from jax import lax
from jax.experimental import pallas as pl
from jax.experimental.pallas import tpu as pltpu
from jax.experimental.pallas import tpu_sc as plsc

N = 10000
E = 320000
IN_DIM = 128
HID = 32
NC_OUT = 16

NCORES = 2
NSUB = 16
NW = NCORES * NSUB
EPT = E // NW
BATCH = 125
NB = EPT // BATCH
NPAD = 10240
NODES_PER_SUB = NPAD // NSUB
NSLOT = 5
DEG_GRP = 10

_MESH = plsc.VectorSubcoreMesh(
    core_axis_name="c", subcore_axis_name="s",
    num_cores=NCORES, num_subcores=NSUB)
_SC_PARAMS = pltpu.CompilerParams(use_tc_tiling_on_sc=False)


@functools.partial(
    pl.kernel,
    out_type=jax.ShapeDtypeStruct((NCORES * NPAD, 8), jnp.float32),
    mesh=_MESH,
    compiler_params=_SC_PARAMS,
    scratch_types=[
        pltpu.VMEM((NB, BATCH), jnp.int32),
        pltpu.VMEM((BATCH, 8), jnp.float32),
        pltpu.VMEM_SHARED((NPAD, 8), jnp.float32),
        pltpu.SemaphoreType.DMA,
    ],
)
def _deg_kernel(dst_hbm, ones_hbm, zeros_hbm, out_hbm, didx, ones_v, acc,
                sem):
    c = lax.axis_index("c")
    s = lax.axis_index("s")
    w = c * NSUB + s
    pltpu.sync_copy(dst_hbm.at[pl.ds(w * NB, NB)], didx)
    pltpu.sync_copy(ones_hbm, ones_v)
    base = s * NODES_PER_SUB
    pltpu.sync_copy(zeros_hbm.at[pl.ds(base, NODES_PER_SUB)],
                    acc.at[pl.ds(base, NODES_PER_SUB)])
    plsc.subcore_barrier()

    def body(i, carry):
        cps = [pltpu.async_copy(ones_v, acc.at[didx.at[i * DEG_GRP + k]],
                                sem, add=True)
               for k in range(DEG_GRP)]
        for cp in cps:
            cp.wait()
        return carry

    lax.fori_loop(0, NB // DEG_GRP, body, 0)
    plsc.subcore_barrier()
    pltpu.sync_copy(acc.at[pl.ds(base, NODES_PER_SUB)],
                    out_hbm.at[pl.ds(c * NPAD + base, NODES_PER_SUB)])


def _make_conv_scatter(feat):
    @functools.partial(
        pl.kernel,
        out_type=jax.ShapeDtypeStruct((NCORES * NPAD, feat), jnp.float32),
        mesh=_MESH,
        compiler_params=_SC_PARAMS,
        scratch_types=[
            pltpu.VMEM((NB, BATCH), jnp.int32),
            pltpu.VMEM((NB, BATCH), jnp.int32),
            pltpu.VMEM((NSLOT, BATCH, feat), jnp.float32),
            pltpu.VMEM_SHARED((NPAD, feat), jnp.float32),
        ] + [pltpu.SemaphoreType.DMA] * NSLOT,
    )
    def conv_scatter(hs_hbm, src_hbm, dst_hbm, zeros_hbm, out_hbm,
                     sidx, didx, rows, acc, *sems):
        c = lax.axis_index("c")
        s = lax.axis_index("s")
        w = c * NSUB + s
        pltpu.sync_copy(src_hbm.at[pl.ds(w * NB, NB)], sidx)
        pltpu.sync_copy(dst_hbm.at[pl.ds(w * NB, NB)], didx)
        base = s * NODES_PER_SUB
        pltpu.sync_copy(zeros_hbm.at[pl.ds(base, NODES_PER_SUB)],
                        acc.at[pl.ds(base, NODES_PER_SUB)])
        plsc.subcore_barrier()

        for k in range(NSLOT):
            pltpu.async_copy(hs_hbm.at[sidx.at[k]], rows.at[k], sems[k])

        def body(i, carry):
            for k in range(NSLOT):
                b = i * NSLOT + k
                pltpu.make_async_copy(hs_hbm.at[sidx.at[0]], rows.at[k],
                                      sems[k]).wait()
                pltpu.sync_copy(rows.at[k], acc.at[didx.at[b]], add=True)

                @pl.when(b + NSLOT < NB)
                def _():
                    pltpu.async_copy(hs_hbm.at[sidx.at[b + NSLOT]],
                                     rows.at[k], sems[k])
            return carry

        lax.fori_loop(0, NB // NSLOT, body, 0)

        plsc.subcore_barrier()
        pltpu.sync_copy(acc.at[pl.ds(base, NODES_PER_SUB)],
                        out_hbm.at[pl.ds(c * NPAD + base, NODES_PER_SUB)])

    return conv_scatter


_conv32 = _make_conv_scatter(HID)
_conv16 = _make_conv_scatter(NC_OUT)


def _mm_body(x_ref, w_ref, o_ref):
    o_ref[...] = jnp.dot(x_ref[...], w_ref[...],
                         preferred_element_type=jnp.float32)


def _scale_body(degp_ref, h_ref, hs_ref, dinv_ref):
    deg8 = degp_ref[0:N] + degp_ref[NPAD:NPAD + N] + 1.0
    dinv8 = lax.rsqrt(deg8)
    dinv_ref[...] = dinv8
    hs_ref[...] = h_ref[...] * dinv8[:, :1]


def _combine_body(p_ref, h_ref, hmlp_ref, dinv_ref, w2_ref, b1_ref, bmlp_ref,
                  g2_ref, g2s_ref):
    d1 = dinv_ref[:, :1]
    psum = p_ref[0:N] + p_ref[NPAD:NPAD + N]
    out1 = d1 * psum + d1 * d1 * h_ref[...] + b1_ref[...]
    x2 = jnp.maximum(out1, 0.0) + hmlp_ref[...] + bmlp_ref[...]
    g2 = jnp.dot(x2, w2_ref[...], preferred_element_type=jnp.float32)
    g2_ref[...] = g2
    g2s_ref[...] = g2 * d1


def _final_body(q_ref, g2_ref, dinv_ref, b2_ref, o_ref):
    d1 = dinv_ref[:, :1]
    qsum = q_ref[0:N] + q_ref[NPAD:NPAD + N]
    out2 = d1 * qsum + d1 * d1 * g2_ref[...] + b2_ref[...]
    m = jnp.max(out2, axis=1, keepdims=True)
    e = jnp.exp(out2 - m)
    o_ref[...] = e / jnp.sum(e, axis=1, keepdims=True)


_mm = pl.pallas_call(
    _mm_body, out_shape=jax.ShapeDtypeStruct((N, 2 * HID), jnp.float32))

_scale = pl.pallas_call(
    _scale_body,
    out_shape=[jax.ShapeDtypeStruct((N, HID), jnp.float32),
               jax.ShapeDtypeStruct((N, 8), jnp.float32)])

_combine = pl.pallas_call(
    _combine_body,
    out_shape=[jax.ShapeDtypeStruct((N, NC_OUT), jnp.float32),
               jax.ShapeDtypeStruct((N, NC_OUT), jnp.float32)])

_final = pl.pallas_call(
    _final_body, out_shape=jax.ShapeDtypeStruct((N, NC_OUT), jnp.float32))


def kernel(x, edge_index, g, A_k, D, Kindices, de, M, I,
           W1, b1, Wmlp, bmlp, W2, b2):
    f32 = jnp.float32
    src = edge_index[0].reshape(NW * NB, BATCH)
    dst = edge_index[1].reshape(NW * NB, BATCH)
    ones8 = jnp.ones((BATCH, 8), f32)
    zeros8 = jnp.zeros((NPAD, 8), f32)
    zeros32 = jnp.zeros((NPAD, HID), f32)
    zeros16 = jnp.zeros((NPAD, NC_OUT), f32)

    degp = _deg_kernel(dst, ones8, zeros8)
    hh = _mm(x, jnp.concatenate([W1, Wmlp], axis=1))
    h = hh[:, :HID]
    hmlp = hh[:, HID:]
    hs, dinv8 = _scale(degp, h)
    p = _conv32(hs, src, dst, zeros32)
    g2, g2s = _combine(p, h, hmlp, dinv8, W2,
                       b1.reshape(1, HID), bmlp.reshape(1, HID))
    q = _conv16(g2s, src, dst, zeros16)
    return _final(q, g2, dinv8, b2.reshape(1, NC_OUT))
